# baseline (speedup 1.0000x reference)
import dataclasses

import jax
import jax.numpy as jnp
from jax import lax
from jax.experimental import pallas as pl
from jax.experimental.pallas import tpu as pltpu
from jax.experimental.pallas import tpu_sc as plsc

R = 32
B = 4
C = 128
N = 100000
V = R * R * R
CH = 4000
NCHUNK = N // CH


def _coords_body(coords_ref, norm_ref, idx_ref):
    c = coords_ref[...]
    norm = jnp.clip(((c + 1.0) / 2.0) * R, 0.0, R - 1.0)
    norm_ref[...] = norm
    v = jnp.round(norm).astype(jnp.int32)
    idx_ref[...] = (v[:, 0] * R + v[:, 1]) * R + v[:, 2]


def _make_sc_body(q):
    def _sc_body(feat_hbm, idx_hbm, out_hbm, g0, g1, cnt_ref,
                 va0, vb0, va1, vb1, ia0, ia1, sem0, sem1):
        wid = lax.axis_index("s") * 2 + lax.axis_index("c")
        row0 = wid * 4

        zeros = jnp.zeros((16,), jnp.float32)
        ones = jnp.ones((16,), jnp.float32)
        ibase = q * N

        def fire_idx(ci, ib, sem):
            pltpu.async_copy(idx_hbm.at[pl.ds(ibase + ci * CH, CH)], ib, sem)

        def drain_idx(ib, sem):
            pltpu.make_async_copy(idx_hbm.at[pl.ds(0, CH)], ib, sem).wait()

        @plsc.parallel_loop(0, V // 16, unroll=8)
        def _(i):
            cnt_ref[pl.ds(i * 16, 16)] = zeros

        def count_from(ib):
            @plsc.parallel_loop(0, CH // 16, unroll=10)
            def _(i):
                iv = ib[pl.ds(i * 16, 16)]
                plsc.addupdate_scatter(cnt_ref, [iv], ones)

        fire_idx(0, ia0, sem0)

        @pl.loop(0, NCHUNK - 1, step=2)
        def _(ci):
            fire_idx(ci + 1, ia1, sem1)
            drain_idx(ia0, sem0)
            count_from(ia0)
            fire_idx(ci + 2, ia0, sem0)
            drain_idx(ia1, sem1)
            count_from(ia1)

        drain_idx(ia0, sem0)
        count_from(ia0)

        @plsc.parallel_loop(0, V // 16, unroll=4)
        def _(i):
            sl = pl.ds(i * 16, 16)
            cnt_ref[sl] = 1.0 / jnp.maximum(cnt_ref[sl], 1.0)

        for p in range(2):
            rowa = row0 + p * 2
            abase = (q * C + rowa) * N
            bbase = abase + N

            @plsc.parallel_loop(0, V // 16, unroll=8)
            def _(i):
                sl = pl.ds(i * 16, 16)
                g0[sl] = zeros
                g1[sl] = zeros

            def fire(ci, va, vb, ib, sem):
                off = ci * CH
                pltpu.async_copy(feat_hbm.at[pl.ds(abase + off, CH)], va, sem)
                pltpu.async_copy(feat_hbm.at[pl.ds(bbase + off, CH)], vb, sem)
                pltpu.async_copy(idx_hbm.at[pl.ds(ibase + off, CH)], ib, sem)

            def drain(va, vb, ib, sem):
                pltpu.make_async_copy(
                    feat_hbm.at[pl.ds(0, CH)], va, sem).wait()
                pltpu.make_async_copy(
                    feat_hbm.at[pl.ds(0, CH)], vb, sem).wait()
                pltpu.make_async_copy(idx_hbm.at[pl.ds(0, CH)], ib, sem).wait()

            def compute(va, vb, ib):
                @plsc.parallel_loop(0, CH // 16, unroll=10)
                def _(i):
                    sl = pl.ds(i * 16, 16)
                    iv = ib[sl]
                    plsc.addupdate_scatter(g0, [iv], va[sl])
                    plsc.addupdate_scatter(g1, [iv], vb[sl])

            fire(0, va0, vb0, ia0, sem0)

            @pl.loop(0, NCHUNK - 1, step=2)
            def _(ci):
                fire(ci + 1, va1, vb1, ia1, sem1)
                drain(va0, vb0, ia0, sem0)
                compute(va0, vb0, ia0)
                fire(ci + 2, va0, vb0, ia0, sem0)
                drain(va1, vb1, ia1, sem1)
                compute(va1, vb1, ia1)

            drain(va0, vb0, ia0, sem0)
            compute(va0, vb0, ia0)

            @plsc.parallel_loop(0, V // 16, unroll=4)
            def _(i):
                sl = pl.ds(i * 16, 16)
                g0[sl] = g0[sl] * cnt_ref[sl]
                g1[sl] = g1[sl] * cnt_ref[sl]

            pltpu.sync_copy(g0, out_hbm.at[pl.ds(rowa * V, V)])
            pltpu.sync_copy(g1, out_hbm.at[pl.ds((rowa + 1) * V, V)])

    return _sc_body


_cp = pltpu.CompilerParams()
if "needs_layout_passes" in pltpu.CompilerParams.__dataclass_fields__:
    _cp = dataclasses.replace(_cp, needs_layout_passes=False)

_sc_calls = [
    pl.kernel(
        _make_sc_body(0),
        out_type=jax.ShapeDtypeStruct((C * V,), jnp.float32),
        mesh=plsc.VectorSubcoreMesh(core_axis_name="c", subcore_axis_name="s"),
        scratch_types=[
            pltpu.VMEM((V,), jnp.float32),
            pltpu.VMEM((V,), jnp.float32),
            pltpu.VMEM((V,), jnp.float32),
            pltpu.VMEM((CH,), jnp.float32),
            pltpu.VMEM((CH,), jnp.float32),
            pltpu.VMEM((CH,), jnp.float32),
            pltpu.VMEM((CH,), jnp.float32),
            pltpu.VMEM((CH,), jnp.int32),
            pltpu.VMEM((CH,), jnp.int32),
            pltpu.SemaphoreType.DMA,
            pltpu.SemaphoreType.DMA,
        ],
        compiler_params=_cp,
    )
    for q in range(B)
]


def kernel(features, coords):
    norm, idx = pl.pallas_call(
        _coords_body,
        out_shape=[
            jax.ShapeDtypeStruct((B, 3, N), jnp.float32),
            jax.ShapeDtypeStruct((B, N), jnp.int32),
        ],
    )(coords)
    outs = [
        _sc_calls[q](features[q].reshape(C * N), idx[q]) for q in range(B)
    ]
    out = jnp.stack(outs).reshape(B, C, R, R, R)
    return out, norm


# speedup vs baseline: 1.0052x; 1.0052x over previous
import dataclasses

import jax
import jax.numpy as jnp
from jax import lax
from jax.experimental import pallas as pl
from jax.experimental.pallas import tpu as pltpu
from jax.experimental.pallas import tpu_sc as plsc

R = 32
B = 4
C = 128
N = 100000
V = R * R * R
CH = 4000
NCHUNK = N // CH


def _coords_body(coords_ref, norm_ref, idx_ref):
    c = coords_ref[...]
    norm = jnp.clip(((c + 1.0) / 2.0) * R, 0.0, R - 1.0)
    norm_ref[...] = norm
    v = jnp.round(norm).astype(jnp.int32)
    idx_ref[...] = (v[:, 0] * R + v[:, 1]) * R + v[:, 2]


def _make_sc_body(q):
    def _sc_body(feat_hbm, idx_hbm, out_hbm, g0, g1, cnt_ref,
                 va0, vb0, va1, vb1, ia0, ia1, sem0, sem1):
        wid = lax.axis_index("s") * 2 + lax.axis_index("c")
        row0 = wid * 4

        zeros = jnp.zeros((16,), jnp.float32)
        ones = jnp.ones((16,), jnp.float32)
        ibase = q * N

        def fire_idx(ci, ib, sem):
            pltpu.async_copy(idx_hbm.at[pl.ds(ibase + ci * CH, CH)], ib, sem)

        def drain_idx(ib, sem):
            pltpu.make_async_copy(idx_hbm.at[pl.ds(0, CH)], ib, sem).wait()

        @plsc.parallel_loop(0, V // 16, unroll=8)
        def _(i):
            cnt_ref[pl.ds(i * 16, 16)] = zeros

        def count_from(ib):
            @plsc.parallel_loop(0, CH // 16, unroll=5)
            def _(i):
                iv = ib[pl.ds(i * 16, 16)]
                plsc.addupdate_scatter(cnt_ref, [iv], ones)

        fire_idx(0, ia0, sem0)

        @pl.loop(0, NCHUNK - 1, step=2)
        def _(ci):
            fire_idx(ci + 1, ia1, sem1)
            drain_idx(ia0, sem0)
            count_from(ia0)
            fire_idx(ci + 2, ia0, sem0)
            drain_idx(ia1, sem1)
            count_from(ia1)

        drain_idx(ia0, sem0)
        count_from(ia0)

        @plsc.parallel_loop(0, V // 16, unroll=4)
        def _(i):
            sl = pl.ds(i * 16, 16)
            cnt_ref[sl] = 1.0 / jnp.maximum(cnt_ref[sl], 1.0)

        for p in range(2):
            rowa = row0 + p * 2
            abase = (q * C + rowa) * N
            bbase = abase + N

            @plsc.parallel_loop(0, V // 16, unroll=8)
            def _(i):
                sl = pl.ds(i * 16, 16)
                g0[sl] = zeros
                g1[sl] = zeros

            def fire(ci, va, vb, ib, sem):
                off = ci * CH
                pltpu.async_copy(feat_hbm.at[pl.ds(abase + off, CH)], va, sem)
                pltpu.async_copy(feat_hbm.at[pl.ds(bbase + off, CH)], vb, sem)
                pltpu.async_copy(idx_hbm.at[pl.ds(ibase + off, CH)], ib, sem)

            def drain(va, vb, ib, sem):
                pltpu.make_async_copy(
                    feat_hbm.at[pl.ds(0, CH)], va, sem).wait()
                pltpu.make_async_copy(
                    feat_hbm.at[pl.ds(0, CH)], vb, sem).wait()
                pltpu.make_async_copy(idx_hbm.at[pl.ds(0, CH)], ib, sem).wait()

            def compute(va, vb, ib):
                @plsc.parallel_loop(0, CH // 16, unroll=5)
                def _(i):
                    sl = pl.ds(i * 16, 16)
                    iv = ib[sl]
                    plsc.addupdate_scatter(g0, [iv], va[sl])
                    plsc.addupdate_scatter(g1, [iv], vb[sl])

            fire(0, va0, vb0, ia0, sem0)

            @pl.loop(0, NCHUNK - 1, step=2)
            def _(ci):
                fire(ci + 1, va1, vb1, ia1, sem1)
                drain(va0, vb0, ia0, sem0)
                compute(va0, vb0, ia0)
                fire(ci + 2, va0, vb0, ia0, sem0)
                drain(va1, vb1, ia1, sem1)
                compute(va1, vb1, ia1)

            drain(va0, vb0, ia0, sem0)
            compute(va0, vb0, ia0)

            @plsc.parallel_loop(0, V // 16, unroll=4)
            def _(i):
                sl = pl.ds(i * 16, 16)
                g0[sl] = g0[sl] * cnt_ref[sl]
                g1[sl] = g1[sl] * cnt_ref[sl]

            pltpu.sync_copy(g0, out_hbm.at[pl.ds(rowa * V, V)])
            pltpu.sync_copy(g1, out_hbm.at[pl.ds((rowa + 1) * V, V)])

    return _sc_body


_cp = pltpu.CompilerParams()
if "needs_layout_passes" in pltpu.CompilerParams.__dataclass_fields__:
    _cp = dataclasses.replace(_cp, needs_layout_passes=False)

_sc_calls = [
    pl.kernel(
        _make_sc_body(0),
        out_type=jax.ShapeDtypeStruct((C * V,), jnp.float32),
        mesh=plsc.VectorSubcoreMesh(core_axis_name="c", subcore_axis_name="s"),
        scratch_types=[
            pltpu.VMEM((V,), jnp.float32),
            pltpu.VMEM((V,), jnp.float32),
            pltpu.VMEM((V,), jnp.float32),
            pltpu.VMEM((CH,), jnp.float32),
            pltpu.VMEM((CH,), jnp.float32),
            pltpu.VMEM((CH,), jnp.float32),
            pltpu.VMEM((CH,), jnp.float32),
            pltpu.VMEM((CH,), jnp.int32),
            pltpu.VMEM((CH,), jnp.int32),
            pltpu.SemaphoreType.DMA,
            pltpu.SemaphoreType.DMA,
        ],
        compiler_params=_cp,
    )
    for q in range(B)
]


def kernel(features, coords):
    norm, idx = pl.pallas_call(
        _coords_body,
        out_shape=[
            jax.ShapeDtypeStruct((B, 3, N), jnp.float32),
            jax.ShapeDtypeStruct((B, N), jnp.int32),
        ],
    )(coords)
    outs = [
        _sc_calls[q](features[q].reshape(C * N), idx[q]) for q in range(B)
    ]
    out = jnp.stack(outs).reshape(B, C, R, R, R)
    return out, norm


# speedup vs baseline: 1.0469x; 1.0415x over previous
import dataclasses

import jax
import jax.numpy as jnp
from jax import lax
from jax.experimental import pallas as pl
from jax.experimental.pallas import tpu as pltpu
from jax.experimental.pallas import tpu_sc as plsc

R = 32
B = 4
C = 128
N = 100000
V = R * R * R
CH = 4000
NCHUNK = N // CH


def _coords_body(coords_ref, norm_ref, idx_ref):
    c = coords_ref[...]
    norm = jnp.clip(((c + 1.0) / 2.0) * R, 0.0, R - 1.0)
    norm_ref[...] = norm
    v = jnp.round(norm).astype(jnp.int32)
    idx_ref[...] = (v[:, 0] * R + v[:, 1]) * R + v[:, 2]


def _make_sc_body(q):
    def _sc_body(feat_hbm, idx_hbm, out_hbm, g0, g1, cnt_ref,
                 va0, vb0, va1, vb1, ia0, ia1, sem0, sem1):
        wid = lax.axis_index("s") * 2 + lax.axis_index("c")
        row0 = wid * 4

        zeros = jnp.zeros((16,), jnp.float32)
        ones = jnp.ones((16,), jnp.float32)
        ibase = q * N

        def fire_idx(ci, ib, sem):
            pltpu.async_copy(idx_hbm.at[pl.ds(ibase + ci * CH, CH)], ib, sem)

        def drain_idx(ib, sem):
            pltpu.make_async_copy(idx_hbm.at[pl.ds(0, CH)], ib, sem).wait()

        @plsc.parallel_loop(0, V // 16, unroll=8)
        def _(i):
            cnt_ref[pl.ds(i * 16, 16)] = zeros

        for p in range(2):
            rowa = row0 + p * 2
            abase = (q * C + rowa) * N
            bbase = abase + N

            @plsc.parallel_loop(0, V // 16, unroll=8)
            def _(i):
                sl = pl.ds(i * 16, 16)
                g0[sl] = zeros
                g1[sl] = zeros

            def fire(ci, va, vb, ib, sem):
                off = ci * CH
                pltpu.async_copy(feat_hbm.at[pl.ds(abase + off, CH)], va, sem)
                pltpu.async_copy(feat_hbm.at[pl.ds(bbase + off, CH)], vb, sem)
                pltpu.async_copy(idx_hbm.at[pl.ds(ibase + off, CH)], ib, sem)

            def drain(va, vb, ib, sem):
                pltpu.make_async_copy(
                    feat_hbm.at[pl.ds(0, CH)], va, sem).wait()
                pltpu.make_async_copy(
                    feat_hbm.at[pl.ds(0, CH)], vb, sem).wait()
                pltpu.make_async_copy(idx_hbm.at[pl.ds(0, CH)], ib, sem).wait()

            def compute(va, vb, ib):
                @plsc.parallel_loop(0, CH // 16, unroll=5)
                def _(i):
                    sl = pl.ds(i * 16, 16)
                    iv = ib[sl]
                    plsc.addupdate_scatter(g0, [iv], va[sl])
                    plsc.addupdate_scatter(g1, [iv], vb[sl])
                    if p == 0:
                        plsc.addupdate_scatter(cnt_ref, [iv], ones)

            fire(0, va0, vb0, ia0, sem0)

            @pl.loop(0, NCHUNK - 1, step=2)
            def _(ci):
                fire(ci + 1, va1, vb1, ia1, sem1)
                drain(va0, vb0, ia0, sem0)
                compute(va0, vb0, ia0)
                fire(ci + 2, va0, vb0, ia0, sem0)
                drain(va1, vb1, ia1, sem1)
                compute(va1, vb1, ia1)

            drain(va0, vb0, ia0, sem0)
            compute(va0, vb0, ia0)

            if p == 0:
                @plsc.parallel_loop(0, V // 16, unroll=4)
                def _(i):
                    sl = pl.ds(i * 16, 16)
                    cnt_ref[sl] = 1.0 / jnp.maximum(cnt_ref[sl], 1.0)

            @plsc.parallel_loop(0, V // 16, unroll=4)
            def _(i):
                sl = pl.ds(i * 16, 16)
                g0[sl] = g0[sl] * cnt_ref[sl]
                g1[sl] = g1[sl] * cnt_ref[sl]

            pltpu.sync_copy(g0, out_hbm.at[pl.ds(rowa * V, V)])
            pltpu.sync_copy(g1, out_hbm.at[pl.ds((rowa + 1) * V, V)])

    return _sc_body


_cp = pltpu.CompilerParams()
if "needs_layout_passes" in pltpu.CompilerParams.__dataclass_fields__:
    _cp = dataclasses.replace(_cp, needs_layout_passes=False)

_sc_calls = [
    pl.kernel(
        _make_sc_body(0),
        out_type=jax.ShapeDtypeStruct((C * V,), jnp.float32),
        mesh=plsc.VectorSubcoreMesh(core_axis_name="c", subcore_axis_name="s"),
        scratch_types=[
            pltpu.VMEM((V,), jnp.float32),
            pltpu.VMEM((V,), jnp.float32),
            pltpu.VMEM((V,), jnp.float32),
            pltpu.VMEM((CH,), jnp.float32),
            pltpu.VMEM((CH,), jnp.float32),
            pltpu.VMEM((CH,), jnp.float32),
            pltpu.VMEM((CH,), jnp.float32),
            pltpu.VMEM((CH,), jnp.int32),
            pltpu.VMEM((CH,), jnp.int32),
            pltpu.SemaphoreType.DMA,
            pltpu.SemaphoreType.DMA,
        ],
        compiler_params=_cp,
    )
    for q in range(B)
]


def kernel(features, coords):
    norm, idx = pl.pallas_call(
        _coords_body,
        out_shape=[
            jax.ShapeDtypeStruct((B, 3, N), jnp.float32),
            jax.ShapeDtypeStruct((B, N), jnp.int32),
        ],
    )(coords)
    outs = [
        _sc_calls[q](features[q].reshape(C * N), idx[q]) for q in range(B)
    ]
    out = jnp.stack(outs).reshape(B, C, R, R, R)
    return out, norm
